# baseline (speedup 1.0000x reference)
import functools

import jax
import jax.numpy as jnp
from jax import lax
from jax.experimental import pallas as pl
from jax.experimental.pallas import tpu as pltpu
from jax.experimental.pallas import tpu_sc as plsc

_NC = 2
_NS = 16
_NW = _NC * _NS
_L = 16

_TAB = 1024
_TAB_LO = -16.0
_TAB_SCALE = _TAB / 32.0


def _ls_table():
    x = _TAB_LO + (jnp.arange(_TAB, dtype=jnp.float32) + 0.5) / _TAB_SCALE
    return jax.nn.log_sigmoid(x)


@functools.lru_cache(maxsize=None)
def _make_sc_loss(B, S, V, D, K):
    BW = B // _NW
    n_groups = BW // _L
    slots = 1 + K

    mesh = plsc.VectorSubcoreMesh(core_axis_name="c", subcore_axis_name="s")

    @functools.partial(
        pl.kernel,
        mesh=mesh,
        compiler_params=pltpu.CompilerParams(needs_layout_passes=False),
        out_type=jax.ShapeDtypeStruct((_NW, _L), jnp.float32),
        scratch_types=[
            pltpu.VMEM((V * D,), jnp.float32),
            pltpu.VMEM((_TAB,), jnp.float32),
            pltpu.VMEM((D, BW), jnp.float32),
            pltpu.VMEM((D, BW), jnp.float32),
            pltpu.VMEM((S, BW), jnp.int32),
        ] + [
            pltpu.VMEM((S, BW), jnp.int32),
        ] * K + [
            pltpu.VMEM((_L,), jnp.float32),
            pltpu.SemaphoreType.DMA,
            pltpu.SemaphoreType.DMA,
        ],
    )
    def sc_loss(ctx_hbm, *args):
        (x_hbm, *neg_hbms) = args[:1 + K]
        (w_hbm, tab_hbm, out_hbm, w_v, t_v, c0_v, c1_v, x_v) = args[1 + K:9 + K]
        neg_vs = list(args[9 + K:9 + 2 * K])
        (a_v, sem0, sem1) = args[9 + 2 * K:]
        wid = lax.axis_index("s") * _NC + lax.axis_index("c")
        b0 = wid * BW
        pltpu.sync_copy(w_hbm, w_v)
        pltpu.sync_copy(tab_hbm, t_v)
        pltpu.sync_copy(x_hbm.at[:, pl.ds(b0, BW)], x_v)
        for nh, nv in zip(neg_hbms, neg_vs):
            pltpu.sync_copy(nh.at[:, pl.ds(b0, BW)], nv)

        lane = lax.iota(jnp.int32, _L)

        def ctx_src(s):
            return ctx_hbm.at[pl.ds(s * D, D), pl.ds(b0, BW)]

        def log_sigmoid(x):
            xc = jnp.minimum(jnp.maximum(x, _TAB_LO), -_TAB_LO)
            u = (xc - _TAB_LO) * _TAB_SCALE
            i = jnp.minimum(u.astype(jnp.int32), _TAB - 1)
            tail = jnp.minimum(x - _TAB_LO, 0.0)
            return plsc.load_gather(t_v, [i]) + tail

        def seq_step(s, c_v, tot):
            sv = jnp.full((_L,), s, jnp.int32)

            def group_body(g, tot_g):
                blane = g * _L + lane
                rowb = [plsc.load_gather(x_v, [sv, blane]) * D]
                for k in range(K):
                    rowb.append(
                        plsc.load_gather(neg_vs[k], [sv, blane]) * D)
                zeros = jnp.zeros((_L,), jnp.float32)
                carry0 = tuple([lane] + [zeros] * slots)

                def d_body(_, carry):
                    rot = carry[0]
                    accs = carry[1:]
                    cv = plsc.load_gather(c_v, [rot, blane])
                    new_accs = []
                    for k in range(slots):
                        wv = plsc.load_gather(w_v, [rowb[k] + rot])
                        new_accs.append(accs[k] + wv * cv)
                    return tuple([(rot + 1) & (D - 1)] + new_accs)

                out = lax.fori_loop(0, D, d_body, carry0, unroll=4)
                accs = out[1:]
                tot_g = tot_g + log_sigmoid(accs[0])
                for k in range(1, slots):
                    tot_g = tot_g + log_sigmoid(-accs[k])
                return tot_g

            return lax.fori_loop(0, n_groups, group_body, tot)

        pltpu.async_copy(ctx_src(0), c0_v, sem0)

        def pair_body(cp, tot):
            s0 = cp * 2
            pltpu.async_copy(ctx_src(s0 + 1), c1_v, sem1)
            pltpu.make_async_copy(ctx_src(s0), c0_v, sem0).wait()
            tot = seq_step(s0, c0_v, tot)

            @pl.when(cp < S // 2 - 1)
            def _():
                pltpu.async_copy(ctx_src(s0 + 2), c0_v, sem0)

            pltpu.make_async_copy(ctx_src(s0 + 1), c1_v, sem1).wait()
            return seq_step(s0 + 1, c1_v, tot)

        tot = lax.fori_loop(0, S // 2, pair_body,
                            jnp.zeros((_L,), jnp.float32))
        a_v[...] = tot
        pltpu.sync_copy(a_v, out_hbm.at[wid])

    return sc_loss


def _tc_reduce_body(s_ref, o_ref):
    o_ref[0, 0] = -jnp.sum(s_ref[...])


def kernel(X, context, W, neg_samples):
    B, S = X.shape
    V, D = W.shape
    K = neg_samples.shape[-1]

    assert B % (_NW * _L) == 0 and S % 2 == 0
    assert D & (D - 1) == 0

    ctx2d = jnp.transpose(context, (1, 2, 0)).reshape(S * D, B)
    x2d = jnp.transpose(X, (1, 0)).astype(jnp.int32)
    negs = [jnp.transpose(neg_samples[:, :, k], (1, 0)).astype(jnp.int32)
            for k in range(K)]

    parts = _make_sc_loss(B, S, V, D, K)(
        ctx2d, x2d, *negs, W.reshape(V * D), _ls_table())

    loss = pl.pallas_call(
        _tc_reduce_body,
        out_shape=jax.ShapeDtypeStruct((1, 1), jnp.float32),
        out_specs=pl.BlockSpec(memory_space=pltpu.SMEM),
    )(parts)
    return loss[0, 0]


# speedup vs baseline: 1.0156x; 1.0156x over previous
import functools

import jax
import jax.numpy as jnp
from jax import lax
from jax.experimental import pallas as pl
from jax.experimental.pallas import tpu as pltpu
from jax.experimental.pallas import tpu_sc as plsc

_NC = 2
_NS = 16
_NW = _NC * _NS
_L = 16

_TAB = 1024
_TAB_LO = -16.0
_TAB_SCALE = _TAB / 32.0


def _ls_table():
    x = _TAB_LO + (jnp.arange(_TAB, dtype=jnp.float32) + 0.5) / _TAB_SCALE
    return jax.nn.log_sigmoid(x)


@functools.lru_cache(maxsize=None)
def _make_sc_loss(B, S, V, D, K):
    BW = B // _NW
    n_groups = BW // _L
    slots = 1 + K

    mesh = plsc.VectorSubcoreMesh(core_axis_name="c", subcore_axis_name="s")

    @functools.partial(
        pl.kernel,
        mesh=mesh,
        compiler_params=pltpu.CompilerParams(needs_layout_passes=False),
        out_type=jax.ShapeDtypeStruct((_NW, _L), jnp.float32),
        scratch_types=[
            pltpu.VMEM((V * D,), jnp.float32),
            pltpu.VMEM((_TAB,), jnp.float32),
            pltpu.VMEM((D, BW), jnp.float32),
            pltpu.VMEM((D, BW), jnp.float32),
            pltpu.VMEM((S, BW), jnp.int32),
            pltpu.VMEM((K * S, BW), jnp.int32),
            pltpu.VMEM((_L,), jnp.float32),
            pltpu.SemaphoreType.DMA,
            pltpu.SemaphoreType.DMA,
        ],
    )
    def sc_loss(ctx_hbm, x_hbm, neg_hbm, w_hbm, tab_hbm, out_hbm, w_v, t_v,
                c0_v, c1_v, x_v, n_v, a_v, sem0, sem1):
        wid = lax.axis_index("s") * _NC + lax.axis_index("c")
        b0 = wid * BW
        pltpu.sync_copy(w_hbm, w_v)
        pltpu.sync_copy(tab_hbm, t_v)
        pltpu.sync_copy(x_hbm.at[:, pl.ds(b0, BW)], x_v)
        pltpu.sync_copy(neg_hbm.at[:, pl.ds(b0, BW)], n_v)

        lane = lax.iota(jnp.int32, _L)

        def ctx_src(s):
            return ctx_hbm.at[pl.ds(s * D, D), pl.ds(b0, BW)]

        def log_sigmoid(x):
            xc = jnp.minimum(jnp.maximum(x, _TAB_LO), -_TAB_LO)
            u = (xc - _TAB_LO) * _TAB_SCALE
            i = jnp.minimum(u.astype(jnp.int32), _TAB - 1)
            tail = jnp.minimum(x - _TAB_LO, 0.0)
            return plsc.load_gather(t_v, [i]) + tail

        def seq_step(s, c_v, tot):
            sv = jnp.full((_L,), s, jnp.int32)

            def group_body(g, tot_g):
                blane = g * _L + lane
                rowb = [plsc.load_gather(x_v, [sv, blane]) * D]
                for k in range(K):
                    rowb.append(
                        plsc.load_gather(n_v, [sv + k * S, blane]) * D)
                zeros = jnp.zeros((_L,), jnp.float32)
                carry0 = tuple([lane] + [zeros] * slots)

                def d_body(_, carry):
                    rot = carry[0]
                    accs = carry[1:]
                    cv = plsc.load_gather(c_v, [rot, blane])
                    new_accs = []
                    for k in range(slots):
                        wv = plsc.load_gather(w_v, [rowb[k] + rot])
                        new_accs.append(accs[k] + wv * cv)
                    return tuple([(rot + 1) & (D - 1)] + new_accs)

                out = lax.fori_loop(0, D, d_body, carry0, unroll=4)
                accs = out[1:]
                tot_g = tot_g + log_sigmoid(accs[0])
                for k in range(1, slots):
                    tot_g = tot_g + log_sigmoid(-accs[k])
                return tot_g

            return plsc.parallel_loop(0, n_groups, carry=tot)(group_body)

        pltpu.async_copy(ctx_src(0), c0_v, sem0)

        def pair_body(cp, tot):
            s0 = cp * 2
            pltpu.async_copy(ctx_src(s0 + 1), c1_v, sem1)
            pltpu.make_async_copy(ctx_src(s0), c0_v, sem0).wait()
            tot = seq_step(s0, c0_v, tot)

            @pl.when(cp < S // 2 - 1)
            def _():
                pltpu.async_copy(ctx_src(s0 + 2), c0_v, sem0)

            pltpu.make_async_copy(ctx_src(s0 + 1), c1_v, sem1).wait()
            return seq_step(s0 + 1, c1_v, tot)

        tot = lax.fori_loop(0, S // 2, pair_body,
                            jnp.zeros((_L,), jnp.float32))
        a_v[...] = tot
        pltpu.sync_copy(a_v, out_hbm.at[wid])

    return sc_loss


def _tc_reduce_body(s_ref, o_ref):
    o_ref[0, 0] = -jnp.sum(s_ref[...])


def kernel(X, context, W, neg_samples):
    B, S = X.shape
    V, D = W.shape
    K = neg_samples.shape[-1]

    assert B % (_NW * _L) == 0 and S % 2 == 0
    assert D & (D - 1) == 0

    ctx2d = jnp.transpose(context, (1, 2, 0)).reshape(S * D, B)
    x2d = jnp.transpose(X, (1, 0)).astype(jnp.int32)
    neg2d = jnp.transpose(neg_samples, (2, 1, 0)).reshape(K * S, B)
    neg2d = neg2d.astype(jnp.int32)

    parts = _make_sc_loss(B, S, V, D, K)(
        ctx2d, x2d, neg2d, W.reshape(V * D), _ls_table())

    loss = pl.pallas_call(
        _tc_reduce_body,
        out_shape=jax.ShapeDtypeStruct((1, 1), jnp.float32),
        out_specs=pl.BlockSpec(memory_space=pltpu.SMEM),
    )(parts)
    return loss[0, 0]


# speedup vs baseline: 1.0163x; 1.0007x over previous
import functools

import jax
import jax.numpy as jnp
from jax import lax
from jax.experimental import pallas as pl
from jax.experimental.pallas import tpu as pltpu
from jax.experimental.pallas import tpu_sc as plsc

_NC = 2
_NS = 16
_NW = _NC * _NS
_L = 16

_TAB = 1024
_TAB_LO = -16.0
_TAB_SCALE = _TAB / 32.0


def _ls_table():
    x = _TAB_LO + (jnp.arange(_TAB, dtype=jnp.float32) + 0.5) / _TAB_SCALE
    return jax.nn.log_sigmoid(x)


@functools.lru_cache(maxsize=None)
def _make_sc_loss(B, S, V, D, K):
    BW = B // _NW
    n_groups = BW // _L
    slots = 1 + K

    mesh = plsc.VectorSubcoreMesh(core_axis_name="c", subcore_axis_name="s")

    @functools.partial(
        pl.kernel,
        mesh=mesh,
        compiler_params=pltpu.CompilerParams(needs_layout_passes=False),
        out_type=jax.ShapeDtypeStruct((_NW, _L), jnp.float32),
        scratch_types=[
            pltpu.VMEM((V * D,), jnp.float32),
            pltpu.VMEM((_TAB,), jnp.float32),
            pltpu.VMEM((D, BW), jnp.float32),
            pltpu.VMEM((D, BW), jnp.float32),
            pltpu.VMEM((S, BW), jnp.int32),
            pltpu.VMEM((K * S, BW), jnp.int32),
            pltpu.VMEM((_L,), jnp.float32),
            pltpu.SemaphoreType.DMA,
            pltpu.SemaphoreType.DMA,
        ],
    )
    def sc_loss(ctx_hbm, x_hbm, neg_hbm, w_hbm, tab_hbm, out_hbm, w_v, t_v,
                c0_v, c1_v, x_v, n_v, a_v, sem0, sem1):
        wid = lax.axis_index("s") * _NC + lax.axis_index("c")
        b0 = wid * BW
        pltpu.sync_copy(w_hbm, w_v)
        pltpu.sync_copy(tab_hbm, t_v)
        pltpu.sync_copy(x_hbm.at[:, pl.ds(b0, BW)], x_v)
        pltpu.sync_copy(neg_hbm.at[:, pl.ds(b0, BW)], n_v)

        lane = lax.iota(jnp.int32, _L)

        def ctx_src(s):
            return ctx_hbm.at[pl.ds(s * D, D), pl.ds(b0, BW)]

        def log_sigmoid(x):
            xc = jnp.minimum(jnp.maximum(x, _TAB_LO), -_TAB_LO)
            u = (xc - _TAB_LO) * _TAB_SCALE
            i = jnp.minimum(u.astype(jnp.int32), _TAB - 1)
            tail = jnp.minimum(x - _TAB_LO, 0.0)
            return plsc.load_gather(t_v, [i]) + tail

        def seq_step(s, c_v, tot):
            sv = jnp.full((_L,), s, jnp.int32)

            def group_body(g, tot_g):
                blane = g * _L + lane
                rowb = [plsc.load_gather(x_v, [sv, blane]) * D]
                for k in range(K):
                    rowb.append(
                        plsc.load_gather(n_v, [sv + k * S, blane]) * D)
                zeros = jnp.zeros((_L,), jnp.float32)
                carry0 = tuple([lane] + [zeros] * slots)

                def d_body(_, carry):
                    rot = carry[0]
                    accs = carry[1:]
                    cv = plsc.load_gather(c_v, [rot, blane])
                    new_accs = []
                    for k in range(slots):
                        wv = plsc.load_gather(w_v, [rowb[k] + rot])
                        new_accs.append(accs[k] + wv * cv)
                    return tuple([(rot + 1) & (D - 1)] + new_accs)

                out = lax.fori_loop(0, D, d_body, carry0, unroll=4)
                accs = out[1:]
                tot_g = tot_g + log_sigmoid(accs[0])
                for k in range(1, slots):
                    tot_g = tot_g + log_sigmoid(-accs[k])
                return tot_g

            return lax.fori_loop(0, n_groups, group_body, tot)

        pltpu.async_copy(ctx_src(0), c0_v, sem0)

        def pair_body(cp, tot):
            s0 = cp * 2
            pltpu.async_copy(ctx_src(s0 + 1), c1_v, sem1)
            pltpu.make_async_copy(ctx_src(s0), c0_v, sem0).wait()
            tot = seq_step(s0, c0_v, tot)

            @pl.when(cp < S // 2 - 1)
            def _():
                pltpu.async_copy(ctx_src(s0 + 2), c0_v, sem0)

            pltpu.make_async_copy(ctx_src(s0 + 1), c1_v, sem1).wait()
            return seq_step(s0 + 1, c1_v, tot)

        tot = lax.fori_loop(0, S // 2, pair_body,
                            jnp.zeros((_L,), jnp.float32))
        a_v[...] = tot
        pltpu.sync_copy(a_v, out_hbm.at[wid])

    return sc_loss


def _tc_reduce_body(s_ref, o_ref):
    o_ref[0, 0] = -jnp.sum(s_ref[...])


def kernel(X, context, W, neg_samples):
    B, S = X.shape
    V, D = W.shape
    K = neg_samples.shape[-1]

    assert B % (_NW * _L) == 0 and S % 2 == 0
    assert D & (D - 1) == 0

    ctx2d = jnp.transpose(context, (1, 2, 0)).reshape(S * D, B)
    x2d = jnp.transpose(X, (1, 0)).astype(jnp.int32)
    neg2d = jnp.transpose(neg_samples, (2, 1, 0)).reshape(K * S, B)
    neg2d = neg2d.astype(jnp.int32)

    parts = _make_sc_loss(B, S, V, D, K)(
        ctx2d, x2d, neg2d, W.reshape(V * D), _ls_table())

    loss = pl.pallas_call(
        _tc_reduce_body,
        out_shape=jax.ShapeDtypeStruct((1, 1), jnp.float32),
        out_specs=pl.BlockSpec(memory_space=pltpu.SMEM),
    )(parts)
    return loss[0, 0]
